# baseline (speedup 1.0000x reference)
import functools
import jax
import jax.numpy as jnp
from jax import lax
from jax.experimental import pallas as pl
from jax.experimental.pallas import tpu as pltpu
from jax.experimental.pallas import tpu_sc as plsc

_NC = 2
_NS = 16
_NW = _NC * _NS
_L = 16
_CHUNK = 16


def _sc_body(rows_per_w, batch, d_model, emb_hbm, pos_hbm, out_hbm, emb_v, pos_v):
    wid = lax.axis_index("s") * _NC + lax.axis_index("c")
    n_chunks = rows_per_w // _CHUNK
    n_dv = d_model // _L

    def chunk_body(k, carry):
        base = wid * rows_per_w + k * _CHUNK
        pltpu.sync_copy(emb_hbm.at[pl.ds(base, _CHUNK)], emb_v)
        pltpu.sync_copy(pos_hbm.at[pl.ds(base, _CHUNK)], pos_v)

        def s_body(s, c2):
            def d_body(j, c3):
                d = j * _L
                p = pos_v[s, pl.ds(d, _L)]
                for b in range(batch):
                    emb_v[s, b, pl.ds(d, _L)] = emb_v[s, b, pl.ds(d, _L)] + p
                return c3

            return lax.fori_loop(0, n_dv, d_body, c2)

        lax.fori_loop(0, _CHUNK, s_body, 0)
        pltpu.sync_copy(emb_v, out_hbm.at[pl.ds(base, _CHUNK)])
        return carry

    lax.fori_loop(0, n_chunks, chunk_body, 0)


def kernel(embedding, pos_table):
    seq_len, batch, d_model = embedding.shape
    rows_per_w = seq_len // _NW
    mesh = plsc.VectorSubcoreMesh(core_axis_name="c", subcore_axis_name="s")
    body = functools.partial(_sc_body, rows_per_w, batch, d_model)
    return pl.kernel(
        body,
        out_type=jax.ShapeDtypeStruct(embedding.shape, embedding.dtype),
        mesh=mesh,
        scratch_types=[
            pltpu.VMEM((_CHUNK, batch, d_model), jnp.float32),
            pltpu.VMEM((_CHUNK, d_model), jnp.float32),
        ],
    )(embedding, pos_table)


# speedup vs baseline: 1.3496x; 1.3496x over previous
import functools
import jax
import jax.numpy as jnp
from jax import lax
from jax.experimental import pallas as pl
from jax.experimental.pallas import tpu as pltpu
from jax.experimental.pallas import tpu_sc as plsc

_NC = 2
_NS = 16
_NW = _NC * _NS
_L = 16
_CHUNK = 8
_NBUF = 3


def _sc_body(rows_per_w, batch, d_model, emb_hbm, pos_hbm, out_hbm, *scratch):
    emb_bufs = scratch[0:_NBUF]
    pos_bufs = scratch[_NBUF:2 * _NBUF]
    emb_sems = scratch[2 * _NBUF:3 * _NBUF]
    pos_sems = scratch[3 * _NBUF:4 * _NBUF]
    out_sems = scratch[4 * _NBUF:5 * _NBUF]

    wid = lax.axis_index("s") * _NC + lax.axis_index("c")
    row0 = wid * rows_per_w
    n_chunks = rows_per_w // _CHUNK
    n_dv = d_model // _L

    def start_in(k):
        b = k % _NBUF
        base = row0 + k * _CHUNK
        h_e = pltpu.async_copy(emb_hbm.at[pl.ds(base, _CHUNK)], emb_bufs[b], emb_sems[b])
        h_p = pltpu.async_copy(pos_hbm.at[pl.ds(base, _CHUNK)], pos_bufs[b], pos_sems[b])
        return h_e, h_p

    def start_out(k):
        b = k % _NBUF
        base = row0 + k * _CHUNK
        return pltpu.async_copy(emb_bufs[b], out_hbm.at[pl.ds(base, _CHUNK)], out_sems[b])

    def compute(k):
        b = k % _NBUF
        emb = emb_bufs[b]
        pos = pos_bufs[b]

        def s_body(s, c):
            def d_body(j, c2):
                d = j * (2 * _L)
                p0 = pos[s, pl.ds(d, _L)]
                p1 = pos[s, pl.ds(d + _L, _L)]
                for bb in range(batch):
                    emb[s, bb, pl.ds(d, _L)] = emb[s, bb, pl.ds(d, _L)] + p0
                    emb[s, bb, pl.ds(d + _L, _L)] = emb[s, bb, pl.ds(d + _L, _L)] + p1
                return c2

            return lax.fori_loop(0, n_dv // 2, d_body, c)

        lax.fori_loop(0, _CHUNK, s_body, 0)

    in_h = {}
    out_h = {}
    in_h[0] = start_in(0)
    if n_chunks > 1:
        in_h[1] = start_in(1)
    for k in range(n_chunks):
        if k + 2 < n_chunks:
            if k - 1 >= 0:
                out_h[k - 1].wait()
            in_h[k + 2] = start_in(k + 2)
        h_e, h_p = in_h[k]
        h_e.wait()
        h_p.wait()
        compute(k)
        out_h[k] = start_out(k)
    for k in range(max(0, n_chunks - 3), n_chunks):
        out_h[k].wait()


def kernel(embedding, pos_table):
    seq_len, batch, d_model = embedding.shape
    rows_per_w = seq_len // _NW
    mesh = plsc.VectorSubcoreMesh(core_axis_name="c", subcore_axis_name="s")
    body = functools.partial(_sc_body, rows_per_w, batch, d_model)
    scratch = (
        [pltpu.VMEM((_CHUNK, batch, d_model), jnp.float32) for _ in range(_NBUF)]
        + [pltpu.VMEM((_CHUNK, d_model), jnp.float32) for _ in range(_NBUF)]
        + [pltpu.SemaphoreType.DMA for _ in range(3 * _NBUF)]
    )
    return pl.kernel(
        body,
        out_type=jax.ShapeDtypeStruct(embedding.shape, embedding.dtype),
        mesh=mesh,
        scratch_types=scratch,
    )(embedding, pos_table)
